# Initial kernel scaffold; baseline (speedup 1.0000x reference)
import jax
import jax.numpy as jnp
from jax.experimental import pallas as pl


def kernel(x, edge_index, neighborhood_values, W, a):
    raise NotImplementedError("write your pallas kernel here")


# speedup vs baseline: 19.7382x; 19.7382x over previous
import dataclasses
import functools

import jax
import jax.numpy as jnp
from jax import lax
from jax.experimental import pallas as pl
from jax.experimental.pallas import tpu as pltpu
from jax.experimental.pallas import tpu_sc as plsc

N = 10000
E = 320000
D = 128
NEG_SLOPE = 0.2

NC = 2
NS = 16
NW = NC * NS
EPT = E // NW
CH = 80
CHB = 2000
NPAD = 10240
RPT = NPAD // NS
G = D // 16

_BLK1 = 1000
_BLK3 = 1000


def _proj_body(x_ref, w_ref, ap_ref, msg_ref, as_ref, ad_ref):
    m = jnp.dot(x_ref[...], w_ref[...], preferred_element_type=jnp.float32)
    msg_ref[...] = m
    al = jnp.dot(m, ap_ref[...], preferred_element_type=jnp.float32)
    as_ref[...] = al[:, 0:1]
    ad_ref[...] = al[:, 1:2]


def _project(x, W, a_pair):
    grid = (N // _BLK1,)
    return pl.pallas_call(
        _proj_body,
        grid=grid,
        in_specs=[
            pl.BlockSpec((_BLK1, D), lambda i: (i, 0)),
            pl.BlockSpec((D, D), lambda i: (0, 0)),
            pl.BlockSpec((D, 2), lambda i: (0, 0)),
        ],
        out_specs=[
            pl.BlockSpec((_BLK1, D), lambda i: (i, 0)),
            pl.BlockSpec((_BLK1, 1), lambda i: (i, 0)),
            pl.BlockSpec((_BLK1, 1), lambda i: (i, 0)),
        ],
        out_shape=[
            jax.ShapeDtypeStruct((N, D), jnp.float32),
            jax.ShapeDtypeStruct((N, 1), jnp.float32),
            jax.ShapeDtypeStruct((N, 1), jnp.float32),
        ],
    )(x, W, a_pair)


_SC_MESH = plsc.VectorSubcoreMesh(
    core_axis_name="c", subcore_axis_name="s", num_cores=NC, num_subcores=NS)

_SC_PARAMS = pltpu.CompilerParams()
if "needs_layout_passes" in pltpu.CompilerParams.__dataclass_fields__:
    _SC_PARAMS = dataclasses.replace(_SC_PARAMS, needs_layout_passes=False)


@functools.partial(
    pl.kernel,
    compiler_params=_SC_PARAMS,
    out_type=(
        jax.ShapeDtypeStruct((NC, NPAD, D), jnp.float32),
        jax.ShapeDtypeStruct((NC, NPAD), jnp.float32),
    ),
    mesh=_SC_MESH,
    scratch_types=[
        pltpu.VMEM_SHARED((NPAD, D), jnp.float32),
        pltpu.VMEM_SHARED((NPAD,), jnp.float32),
        pltpu.VMEM((N,), jnp.float32),
        pltpu.VMEM((N,), jnp.float32),
        pltpu.VMEM((CHB,), jnp.int32),
        pltpu.VMEM((CHB,), jnp.int32),
        pltpu.VMEM((CHB,), jnp.float32),
        pltpu.VMEM((2, CH), jnp.float32),
        pltpu.VMEM((2, CH), jnp.float32),
        pltpu.VMEM((2, CH), jnp.int32),
        pltpu.VMEM((2, CH, D), jnp.float32),
        pltpu.SemaphoreType.DMA,
        pltpu.SemaphoreType.DMA,
        pltpu.SemaphoreType.DMA,
        pltpu.SemaphoreType.DMA,
        pltpu.SemaphoreType.DMA,
        pltpu.SemaphoreType.DMA,
    ],
)
def _edge_sweep(msg_hbm, asrc_hbm, adst_hbm, src_hbm, dst_hbm, nv_hbm,
                acc_out, rs_out, acc_sh, rs_sh, as_v, ad_v, srcb, dstb,
                nvb, e2, w2, sidx2, rows2, gsem0, gsem1, asem0, asem1,
                rsem0, rsem1):
    c = lax.axis_index("c")
    s = lax.axis_index("s")
    wid = c * NS + s
    ebase = wid * EPT
    rbase = s * RPT

    pltpu.sync_copy(asrc_hbm, as_v)
    pltpu.sync_copy(adst_hbm, ad_v)

    zero16 = jnp.zeros((16,), jnp.float32)

    @pl.loop(0, CH)
    def _zero_rows(k):
        for g in range(G):
            rows2[0, k, pl.ds(g * 16, 16)] = zero16

    for q in range(RPT // CH):
        pltpu.sync_copy(rows2.at[0], acc_sh.at[pl.ds(rbase + q * CH, CH)])
    for q in range(RPT // D):
        pltpu.sync_copy(rows2.at[0, 0], rs_sh.at[pl.ds(rbase + q * D, D)])
    plsc.subcore_barrier()

    gsems = (gsem0, gsem1)
    asems = (asem0, asem1)
    rsems = (rsem0, rsem1)

    def _gather_start(eb, b):
        pltpu.async_copy(
            msg_hbm.at[dstb.at[pl.ds(eb, CH)]], rows2.at[b], gsems[b])

    def _gather_wait(b):
        pltpu.make_async_copy(
            msg_hbm.at[dstb.at[pl.ds(0, CH)]], rows2.at[b], gsems[b]).wait()

    def _logits(eb, b):
        eb_ = e2.at[b]
        wb_ = w2.at[b]
        sb_ = sidx2.at[b]
        for v in range(CH // 16):
            sl16 = pl.ds(v * 16, 16)
            sv = srcb[pl.ds(eb + v * 16, 16)]
            dv = dstb[pl.ds(eb + v * 16, 16)]
            t = plsc.load_gather(as_v, [sv]) + plsc.load_gather(ad_v, [dv])
            e = jnp.where(t > 0.0, t, NEG_SLOPE * t)
            eb_[sl16] = e
            wb_[sl16] = e * nvb[pl.ds(eb + v * 16, 16)]
            sb_[sl16] = sv

    def _scale(b):

        @pl.loop(0, CH, step=2)
        def _scale_k(k):
            wv0 = plsc.load_gather(w2.at[b], [jnp.full((16,), k, jnp.int32)])
            wv1 = plsc.load_gather(w2.at[b],
                                   [jnp.full((16,), k + 1, jnp.int32)])
            for g in range(G):
                sl = pl.ds(g * 16, 16)
                rows2[b, k, sl] = rows2[b, k, sl] * wv0
            for g in range(G):
                sl = pl.ds(g * 16, 16)
                rows2[b, k + 1, sl] = rows2[b, k + 1, sl] * wv1

    def _scatter_start(b):
        pltpu.async_copy(rows2.at[b], acc_sh.at[sidx2.at[b]], asems[b],
                         add=True)
        pltpu.async_copy(e2.at[b], rs_sh.at[sidx2.at[b]], rsems[b], add=True)

    def _scatter_wait(b):
        pltpu.make_async_copy(rows2.at[b], acc_sh.at[sidx2.at[b]],
                              asems[b]).wait()
        pltpu.make_async_copy(e2.at[b], rs_sh.at[sidx2.at[b]],
                              rsems[b]).wait()

    def _pair_tail(c):
        _gather_start(c * CH + CH, 1)
        _logits(c * CH + CH, 1)
        _gather_wait(0)
        _scale(0)
        _scatter_start(0)
        _gather_wait(1)
        _scale(1)
        _scatter_start(1)
        _scatter_wait(0)
        _gather_start(c * CH + 2 * CH, 0)
        _logits(c * CH + 2 * CH, 0)

    NCHR = CHB // CH

    @pl.loop(0, EPT, step=CHB)
    def _round(r0):
        pltpu.sync_copy(src_hbm.at[pl.ds(ebase + r0, CHB)], srcb)
        pltpu.sync_copy(dst_hbm.at[pl.ds(ebase + r0, CHB)], dstb)
        pltpu.sync_copy(nv_hbm.at[pl.ds(ebase + r0, CHB)], nvb)

        _gather_start(0, 0)
        _logits(0, 0)
        _pair_tail(0)

        @pl.loop(2, NCHR - 1, step=2)
        def _pair(c):
            _scatter_wait(1)
            _pair_tail(c)

        _scatter_wait(1)
        _gather_wait(0)
        _scale(0)
        _scatter_start(0)
        _scatter_wait(0)

    plsc.subcore_barrier()

    pltpu.sync_copy(acc_sh.at[pl.ds(rbase, RPT)],
                    acc_out.at[c, pl.ds(rbase, RPT)])
    pltpu.sync_copy(rs_sh.at[pl.ds(rbase, RPT)],
                    rs_out.at[c, pl.ds(rbase, RPT)])


def _finish_body(acc_ref, rs_ref, out_ref):
    acc = acc_ref[0] + acc_ref[1]
    rs = rs_ref[0] + rs_ref[1]
    out_ref[...] = jnp.where(rs != 0.0, acc / rs, 0.0)


def _finish(acc, rs3):
    grid = (N // _BLK3,)
    return pl.pallas_call(
        _finish_body,
        grid=grid,
        in_specs=[
            pl.BlockSpec((NC, _BLK3, D), lambda i: (0, i, 0)),
            pl.BlockSpec((NC, _BLK3, 1), lambda i: (0, i, 0)),
        ],
        out_specs=pl.BlockSpec((_BLK3, D), lambda i: (i, 0)),
        out_shape=jax.ShapeDtypeStruct((N, D), jnp.float32),
    )(acc, rs3)


def kernel(x, edge_index, neighborhood_values, W, a):
    a_pair = jnp.concatenate([a[:D], a[D:]], axis=1)
    src = edge_index[0]
    dst = edge_index[1]
    message, al_s, al_d = _project(x, W, a_pair)
    acc, rs = _edge_sweep(message, al_s.reshape(N), al_d.reshape(N),
                          src, dst, neighborhood_values)
    return _finish(acc, rs.reshape(NC, NPAD, 1))


# speedup vs baseline: 20.2134x; 1.0241x over previous
import dataclasses
import functools

import jax
import jax.numpy as jnp
from jax import lax
from jax.experimental import pallas as pl
from jax.experimental.pallas import tpu as pltpu
from jax.experimental.pallas import tpu_sc as plsc

N = 10000
E = 320000
D = 128
NEG_SLOPE = 0.2

NC = 2
NS = 16
NW = NC * NS
EPT = E // NW
CH = 80
CHB = 2000
NPAD = 10240
RPT = NPAD // NS
G = D // 16

_BLK1 = 1000
_BLK3 = 1000


def _proj_body(x_ref, w_ref, ap_ref, msg_ref, as_ref, ad_ref):
    m = jnp.dot(x_ref[...], w_ref[...], preferred_element_type=jnp.float32)
    msg_ref[...] = m
    al = jnp.dot(m, ap_ref[...], preferred_element_type=jnp.float32)
    as_ref[...] = al[:, 0:1]
    ad_ref[...] = al[:, 1:2]


def _project(x, W, a_pair):
    grid = (N // _BLK1,)
    return pl.pallas_call(
        _proj_body,
        grid=grid,
        in_specs=[
            pl.BlockSpec((_BLK1, D), lambda i: (i, 0)),
            pl.BlockSpec((D, D), lambda i: (0, 0)),
            pl.BlockSpec((D, 2), lambda i: (0, 0)),
        ],
        out_specs=[
            pl.BlockSpec((_BLK1, D), lambda i: (i, 0)),
            pl.BlockSpec((_BLK1, 1), lambda i: (i, 0)),
            pl.BlockSpec((_BLK1, 1), lambda i: (i, 0)),
        ],
        out_shape=[
            jax.ShapeDtypeStruct((N, D), jnp.float32),
            jax.ShapeDtypeStruct((N, 1), jnp.float32),
            jax.ShapeDtypeStruct((N, 1), jnp.float32),
        ],
    )(x, W, a_pair)


_SC_MESH = plsc.VectorSubcoreMesh(
    core_axis_name="c", subcore_axis_name="s", num_cores=NC, num_subcores=NS)

_SC_PARAMS = pltpu.CompilerParams()
if "needs_layout_passes" in pltpu.CompilerParams.__dataclass_fields__:
    _SC_PARAMS = dataclasses.replace(_SC_PARAMS, needs_layout_passes=False)


@functools.partial(
    pl.kernel,
    compiler_params=_SC_PARAMS,
    out_type=(
        jax.ShapeDtypeStruct((NC, NPAD, D), jnp.float32),
        jax.ShapeDtypeStruct((NC, NPAD), jnp.float32),
    ),
    mesh=_SC_MESH,
    scratch_types=[
        pltpu.VMEM_SHARED((NPAD, D), jnp.float32),
        pltpu.VMEM_SHARED((NPAD,), jnp.float32),
        pltpu.VMEM((N,), jnp.float32),
        pltpu.VMEM((N,), jnp.float32),
        pltpu.VMEM((CHB,), jnp.int32),
        pltpu.VMEM((CHB,), jnp.int32),
        pltpu.VMEM((CHB,), jnp.float32),
        pltpu.VMEM((2, CH), jnp.float32),
        pltpu.VMEM((2, CH), jnp.float32),
        pltpu.VMEM((2, CH), jnp.int32),
        pltpu.VMEM((2, CH, D), jnp.float32),
        pltpu.SemaphoreType.DMA,
        pltpu.SemaphoreType.DMA,
        pltpu.SemaphoreType.DMA,
        pltpu.SemaphoreType.DMA,
        pltpu.SemaphoreType.DMA,
        pltpu.SemaphoreType.DMA,
    ],
)
def _edge_sweep(msg_hbm, asrc_hbm, adst_hbm, src_hbm, dst_hbm, nv_hbm,
                acc_out, rs_out, acc_sh, rs_sh, as_v, ad_v, srcb, dstb,
                nvb, e2, w2, sidx2, rows2, gsem0, gsem1, asem0, asem1,
                rsem0, rsem1):
    c = lax.axis_index("c")
    s = lax.axis_index("s")
    wid = c * NS + s
    ebase = wid * EPT
    rbase = s * RPT

    pltpu.sync_copy(asrc_hbm, as_v)
    pltpu.sync_copy(adst_hbm, ad_v)

    zero16 = jnp.zeros((16,), jnp.float32)

    @pl.loop(0, CH)
    def _zero_rows(k):
        for g in range(G):
            rows2[0, k, pl.ds(g * 16, 16)] = zero16

    for q in range(RPT // CH):
        pltpu.sync_copy(rows2.at[0], acc_sh.at[pl.ds(rbase + q * CH, CH)])
    for q in range(RPT // D):
        pltpu.sync_copy(rows2.at[0, 0], rs_sh.at[pl.ds(rbase + q * D, D)])
    plsc.subcore_barrier()

    gsems = (gsem0, gsem1)
    asems = (asem0, asem1)
    rsems = (rsem0, rsem1)

    def _gather_start(eb, b):
        pltpu.async_copy(
            msg_hbm.at[dstb.at[pl.ds(eb, CH)]], rows2.at[b], gsems[b])

    def _gather_wait(b):
        pltpu.make_async_copy(
            msg_hbm.at[dstb.at[pl.ds(0, CH)]], rows2.at[b], gsems[b]).wait()

    def _logits(eb, b):
        eb_ = e2.at[b]
        wb_ = w2.at[b]
        sb_ = sidx2.at[b]
        for v in range(CH // 16):
            sl16 = pl.ds(v * 16, 16)
            sv = srcb[pl.ds(eb + v * 16, 16)]
            dv = dstb[pl.ds(eb + v * 16, 16)]
            t = plsc.load_gather(as_v, [sv]) + plsc.load_gather(ad_v, [dv])
            e = jnp.where(t > 0.0, t, NEG_SLOPE * t)
            eb_[sl16] = e
            wb_[sl16] = e * nvb[pl.ds(eb + v * 16, 16)]
            sb_[sl16] = sv

    def _scale(b):

        @pl.loop(0, CH, step=4)
        def _scale_k(k):
            wvs = [plsc.load_gather(w2.at[b],
                                    [jnp.full((16,), k + u, jnp.int32)])
                   for u in range(4)]
            for u in range(4):
                for g in range(G):
                    sl = pl.ds(g * 16, 16)
                    rows2[b, k + u, sl] = rows2[b, k + u, sl] * wvs[u]

    def _scatter_start(b):
        pltpu.async_copy(rows2.at[b], acc_sh.at[sidx2.at[b]], asems[b],
                         add=True)
        pltpu.async_copy(e2.at[b], rs_sh.at[sidx2.at[b]], rsems[b], add=True)

    def _scatter_wait(b):
        pltpu.make_async_copy(rows2.at[b], acc_sh.at[sidx2.at[b]],
                              asems[b]).wait()
        pltpu.make_async_copy(e2.at[b], rs_sh.at[sidx2.at[b]],
                              rsems[b]).wait()

    def _pair_tail(c):
        _gather_start(c * CH + CH, 1)
        _logits(c * CH + CH, 1)
        _gather_wait(0)
        _scale(0)
        _scatter_start(0)
        _gather_wait(1)
        _scale(1)
        _scatter_start(1)
        _scatter_wait(0)
        _gather_start(c * CH + 2 * CH, 0)
        _logits(c * CH + 2 * CH, 0)

    NCHR = CHB // CH

    @pl.loop(0, EPT, step=CHB)
    def _round(r0):
        pltpu.sync_copy(src_hbm.at[pl.ds(ebase + r0, CHB)], srcb)
        pltpu.sync_copy(dst_hbm.at[pl.ds(ebase + r0, CHB)], dstb)
        pltpu.sync_copy(nv_hbm.at[pl.ds(ebase + r0, CHB)], nvb)

        _gather_start(0, 0)
        _logits(0, 0)
        _pair_tail(0)

        @pl.loop(2, NCHR - 1, step=2)
        def _pair(c):
            _scatter_wait(1)
            _pair_tail(c)

        _scatter_wait(1)
        _gather_wait(0)
        _scale(0)
        _scatter_start(0)
        _scatter_wait(0)

    plsc.subcore_barrier()

    pltpu.sync_copy(acc_sh.at[pl.ds(rbase, RPT)],
                    acc_out.at[c, pl.ds(rbase, RPT)])
    pltpu.sync_copy(rs_sh.at[pl.ds(rbase, RPT)],
                    rs_out.at[c, pl.ds(rbase, RPT)])


def _finish_body(acc_ref, rs_ref, out_ref):
    acc = acc_ref[0] + acc_ref[1]
    rs = rs_ref[0] + rs_ref[1]
    out_ref[...] = jnp.where(rs != 0.0, acc / rs, 0.0)


def _finish(acc, rs3):
    grid = (N // _BLK3,)
    return pl.pallas_call(
        _finish_body,
        grid=grid,
        in_specs=[
            pl.BlockSpec((NC, _BLK3, D), lambda i: (0, i, 0)),
            pl.BlockSpec((NC, _BLK3, 1), lambda i: (0, i, 0)),
        ],
        out_specs=pl.BlockSpec((_BLK3, D), lambda i: (i, 0)),
        out_shape=jax.ShapeDtypeStruct((N, D), jnp.float32),
    )(acc, rs3)


def kernel(x, edge_index, neighborhood_values, W, a):
    a_pair = jnp.concatenate([a[:D], a[D:]], axis=1)
    src = edge_index[0]
    dst = edge_index[1]
    message, al_s, al_d = _project(x, W, a_pair)
    acc, rs = _edge_sweep(message, al_s.reshape(N), al_d.reshape(N),
                          src, dst, neighborhood_values)
    return _finish(acc, rs.reshape(NC, NPAD, 1))


# speedup vs baseline: 20.2198x; 1.0003x over previous
import dataclasses
import functools

import jax
import jax.numpy as jnp
from jax import lax
from jax.experimental import pallas as pl
from jax.experimental.pallas import tpu as pltpu
from jax.experimental.pallas import tpu_sc as plsc

N = 10000
E = 320000
D = 128
NEG_SLOPE = 0.2

NC = 2
NS = 16
NW = NC * NS
EPT = E // NW
CH = 80
CHB = 2000
NPAD = 10240
RPT = NPAD // NS
G = D // 16

_BLK1 = 1000
_BLK3 = 1000


def _proj_body(x_ref, w_ref, ap_ref, msg_ref, as_ref, ad_ref):
    m = jnp.dot(x_ref[...], w_ref[...], preferred_element_type=jnp.float32)
    msg_ref[...] = m
    al = jnp.dot(m, ap_ref[...], preferred_element_type=jnp.float32)
    as_ref[...] = al[:, 0:1]
    ad_ref[...] = al[:, 1:2]


def _project(x, W, a_pair):
    grid = (N // _BLK1,)
    return pl.pallas_call(
        _proj_body,
        grid=grid,
        in_specs=[
            pl.BlockSpec((_BLK1, D), lambda i: (i, 0)),
            pl.BlockSpec((D, D), lambda i: (0, 0)),
            pl.BlockSpec((D, 2), lambda i: (0, 0)),
        ],
        out_specs=[
            pl.BlockSpec((_BLK1, D), lambda i: (i, 0)),
            pl.BlockSpec((_BLK1, 1), lambda i: (i, 0)),
            pl.BlockSpec((_BLK1, 1), lambda i: (i, 0)),
        ],
        out_shape=[
            jax.ShapeDtypeStruct((N, D), jnp.float32),
            jax.ShapeDtypeStruct((N, 1), jnp.float32),
            jax.ShapeDtypeStruct((N, 1), jnp.float32),
        ],
    )(x, W, a_pair)


def _sc_params():
    p = pltpu.CompilerParams()
    if "needs_layout_passes" in pltpu.CompilerParams.__dataclass_fields__:
        p = dataclasses.replace(p, needs_layout_passes=False)
    return p


@functools.cache
def _edge_sweep_kernel():
    return pl.kernel(
        _edge_sweep,
        compiler_params=_sc_params(),
        out_type=(
            jax.ShapeDtypeStruct((NC, NPAD, D), jnp.float32),
            jax.ShapeDtypeStruct((NC, NPAD), jnp.float32),
        ),
        mesh=plsc.VectorSubcoreMesh(core_axis_name="c", subcore_axis_name="s",
                                    num_cores=NC, num_subcores=NS),
        scratch_types=[
            pltpu.VMEM_SHARED((NPAD, D), jnp.float32),
            pltpu.VMEM_SHARED((NPAD,), jnp.float32),
            pltpu.VMEM((N,), jnp.float32),
            pltpu.VMEM((N,), jnp.float32),
            pltpu.VMEM((CHB,), jnp.int32),
            pltpu.VMEM((CHB,), jnp.int32),
            pltpu.VMEM((CHB,), jnp.float32),
            pltpu.VMEM((2, CH), jnp.float32),
            pltpu.VMEM((2, CH), jnp.float32),
            pltpu.VMEM((2, CH), jnp.int32),
            pltpu.VMEM((2, CH, D), jnp.float32),
            pltpu.SemaphoreType.DMA,
            pltpu.SemaphoreType.DMA,
            pltpu.SemaphoreType.DMA,
            pltpu.SemaphoreType.DMA,
            pltpu.SemaphoreType.DMA,
            pltpu.SemaphoreType.DMA,
        ],
    )


def _edge_sweep(msg_hbm, asrc_hbm, adst_hbm, src_hbm, dst_hbm, nv_hbm,
                acc_out, rs_out, acc_sh, rs_sh, as_v, ad_v, srcb, dstb,
                nvb, e2, w2, sidx2, rows2, gsem0, gsem1, asem0, asem1,
                rsem0, rsem1):
    c = lax.axis_index("c")
    s = lax.axis_index("s")
    wid = c * NS + s
    ebase = wid * EPT
    rbase = s * RPT

    pltpu.sync_copy(asrc_hbm, as_v)
    pltpu.sync_copy(adst_hbm, ad_v)

    zero16 = jnp.zeros((16,), jnp.float32)

    @pl.loop(0, CH)
    def _zero_rows(k):
        for g in range(G):
            rows2[0, k, pl.ds(g * 16, 16)] = zero16

    for q in range(RPT // CH):
        pltpu.sync_copy(rows2.at[0], acc_sh.at[pl.ds(rbase + q * CH, CH)])
    for q in range(RPT // D):
        pltpu.sync_copy(rows2.at[0, 0], rs_sh.at[pl.ds(rbase + q * D, D)])
    plsc.subcore_barrier()

    gsems = (gsem0, gsem1)
    asems = (asem0, asem1)
    rsems = (rsem0, rsem1)

    def _gather_start(eb, b):
        pltpu.async_copy(
            msg_hbm.at[dstb.at[pl.ds(eb, CH)]], rows2.at[b], gsems[b])

    def _gather_wait(b):
        pltpu.make_async_copy(
            msg_hbm.at[dstb.at[pl.ds(0, CH)]], rows2.at[b], gsems[b]).wait()

    def _logits(eb, b):
        eb_ = e2.at[b]
        wb_ = w2.at[b]
        sb_ = sidx2.at[b]
        for v in range(CH // 16):
            sl16 = pl.ds(v * 16, 16)
            sv = srcb[pl.ds(eb + v * 16, 16)]
            dv = dstb[pl.ds(eb + v * 16, 16)]
            t = plsc.load_gather(as_v, [sv]) + plsc.load_gather(ad_v, [dv])
            e = jnp.where(t > 0.0, t, NEG_SLOPE * t)
            eb_[sl16] = e
            wb_[sl16] = e * nvb[pl.ds(eb + v * 16, 16)]
            sb_[sl16] = sv

    def _scale(b):

        @pl.loop(0, CH, step=4)
        def _scale_k(k):
            wvs = [plsc.load_gather(w2.at[b],
                                    [jnp.full((16,), k + u, jnp.int32)])
                   for u in range(4)]
            for u in range(4):
                for g in range(G):
                    sl = pl.ds(g * 16, 16)
                    rows2[b, k + u, sl] = rows2[b, k + u, sl] * wvs[u]

    def _scatter_start(b):
        pltpu.async_copy(rows2.at[b], acc_sh.at[sidx2.at[b]], asems[b],
                         add=True)
        pltpu.async_copy(e2.at[b], rs_sh.at[sidx2.at[b]], rsems[b], add=True)

    def _scatter_wait(b):
        pltpu.make_async_copy(rows2.at[b], acc_sh.at[sidx2.at[b]],
                              asems[b]).wait()
        pltpu.make_async_copy(e2.at[b], rs_sh.at[sidx2.at[b]],
                              rsems[b]).wait()

    def _pair_tail(c):
        _gather_start(c * CH + CH, 1)
        _logits(c * CH + CH, 1)
        _gather_wait(0)
        _scale(0)
        _scatter_start(0)
        _gather_wait(1)
        _scale(1)
        _scatter_start(1)
        _scatter_wait(0)
        _gather_start(c * CH + 2 * CH, 0)
        _logits(c * CH + 2 * CH, 0)

    NCHR = CHB // CH

    @pl.loop(0, EPT, step=CHB)
    def _round(r0):
        pltpu.sync_copy(src_hbm.at[pl.ds(ebase + r0, CHB)], srcb)
        pltpu.sync_copy(dst_hbm.at[pl.ds(ebase + r0, CHB)], dstb)
        pltpu.sync_copy(nv_hbm.at[pl.ds(ebase + r0, CHB)], nvb)

        _gather_start(0, 0)
        _logits(0, 0)
        _pair_tail(0)

        @pl.loop(2, NCHR - 1, step=2)
        def _pair(c):
            _scatter_wait(1)
            _pair_tail(c)

        _scatter_wait(1)
        _gather_wait(0)
        _scale(0)
        _scatter_start(0)
        _scatter_wait(0)

    plsc.subcore_barrier()

    pltpu.sync_copy(acc_sh.at[pl.ds(rbase, RPT)],
                    acc_out.at[c, pl.ds(rbase, RPT)])
    pltpu.sync_copy(rs_sh.at[pl.ds(rbase, RPT)],
                    rs_out.at[c, pl.ds(rbase, RPT)])


def _finish_body(acc_ref, rs_ref, out_ref):
    acc = acc_ref[0] + acc_ref[1]
    rs = rs_ref[0] + rs_ref[1]
    out_ref[...] = jnp.where(rs != 0.0, acc / rs, 0.0)


def _finish(acc, rs3):
    grid = (N // _BLK3,)
    return pl.pallas_call(
        _finish_body,
        grid=grid,
        in_specs=[
            pl.BlockSpec((NC, _BLK3, D), lambda i: (0, i, 0)),
            pl.BlockSpec((NC, _BLK3, 1), lambda i: (0, i, 0)),
        ],
        out_specs=pl.BlockSpec((_BLK3, D), lambda i: (i, 0)),
        out_shape=jax.ShapeDtypeStruct((N, D), jnp.float32),
    )(acc, rs3)


def kernel(x, edge_index, neighborhood_values, W, a):
    a_pair = jnp.concatenate([a[:D], a[D:]], axis=1)
    src = edge_index[0]
    dst = edge_index[1]
    message, al_s, al_d = _project(x, W, a_pair)
    acc, rs = _edge_sweep_kernel()(message, al_s.reshape(N), al_d.reshape(N),
                                   src, dst, neighborhood_values)
    return _finish(acc, rs.reshape(NC, NPAD, 1))
